# baseline (speedup 1.0000x reference)
import functools

import jax
import jax.numpy as jnp
from jax import lax
from jax.experimental import pallas as pl
from jax.experimental.pallas import tpu as pltpu
from jax.experimental.pallas import tpu_sc as plsc

MAX_WORD_LEN = 16
CHAR_VOCAB = 128
CHAR_DIM = 64
EMBED_SIZE = 128

V_BLOCK = 4000
CHUNK = 128


def _table_body(seqs_ref, lens_ref, ce_ref, w_ref, b_ref, out_ref):
    L = MAX_WORD_LEN
    seqs = seqs_ref[...]
    lens = lens_ref[...].reshape(V_BLOCK, 1)
    lens_f = lens.astype(jnp.float32)
    ones_l = jnp.ones((1, L), jnp.float32)
    lens_b = jnp.dot(lens_f, ones_l,
                     preferred_element_type=jnp.float32)
    pos = lax.broadcasted_iota(jnp.int32, (1, L), 1).astype(jnp.float32)
    sm = jnp.where(pos < lens_b, seqs.astype(jnp.float32),
                   float(CHAR_VOCAB)).astype(jnp.bfloat16)
    col = lax.broadcasted_iota(jnp.int32, (L, L * CHAR_VOCAB), 1)
    row = lax.broadcasted_iota(jnp.int32, (L, L * CHAR_VOCAB), 0)
    p = (col // CHAR_VOCAB == row).astype(jnp.bfloat16)
    ch_all = jnp.dot(sm, p,
                     preferred_element_type=jnp.float32)
    cvec = lax.broadcasted_iota(
        jnp.int32, (1, CHAR_VOCAB), 1).astype(jnp.float32)
    counts_bf = jnp.zeros((V_BLOCK, CHAR_VOCAB), jnp.bfloat16)
    for l in range(L):
        sl = ch_all[:, l * CHAR_VOCAB:(l + 1) * CHAR_VOCAB]
        counts_bf = counts_bf + (sl == cvec).astype(jnp.bfloat16)
    counts = counts_bf.astype(jnp.float32)
    inv1 = 1.0 / lens_f
    inv_b = jnp.dot(inv1, jnp.ones((1, CHAR_DIM), jnp.float32),
                    preferred_element_type=jnp.float32)
    pooled = jnp.dot(counts, ce_ref[...],
                     preferred_element_type=jnp.float32) * inv_b
    out_ref[...] = jnp.dot(pooled, w_ref[...],
                           preferred_element_type=jnp.float32) + b_ref[...]


@functools.lru_cache(maxsize=None)
def _make_table_builder(vocab):
    grid = vocab // V_BLOCK
    return pl.pallas_call(
        _table_body,
        grid=(grid,),
        in_specs=[
            pl.BlockSpec((V_BLOCK, MAX_WORD_LEN), lambda i: (i, 0)),
            pl.BlockSpec((1, 1, V_BLOCK), lambda i: (i, 0, 0)),
            pl.BlockSpec((CHAR_VOCAB, CHAR_DIM), lambda i: (0, 0)),
            pl.BlockSpec((CHAR_DIM, EMBED_SIZE), lambda i: (0, 0)),
            pl.BlockSpec((1, EMBED_SIZE), lambda i: (0, 0)),
        ],
        out_specs=pl.BlockSpec((V_BLOCK, EMBED_SIZE), lambda i: (i, 0)),
        out_shape=jax.ShapeDtypeStruct((vocab, EMBED_SIZE), jnp.float32),
    )


@functools.lru_cache(maxsize=None)
def _make_gather(batch, hist, vocab):
    info = plsc.get_sparse_core_info()
    nw = info.num_cores * info.num_subcores
    rows_per_w = batch // nw
    assert batch % nw == 0 and rows_per_w % 8 == 0 and hist % 2 == 0
    mesh = plsc.VectorSubcoreMesh(core_axis_name="c", subcore_axis_name="s")

    @functools.partial(
        pl.kernel,
        mesh=mesh,
        out_type=jax.ShapeDtypeStruct((hist, batch, EMBED_SIZE), jnp.float32),
        scratch_types=[
            pltpu.VMEM((hist, rows_per_w), jnp.int32),
            pltpu.VMEM((rows_per_w, EMBED_SIZE), jnp.float32),
            pltpu.VMEM((rows_per_w, EMBED_SIZE), jnp.float32),
            pltpu.SemaphoreType.DMA,
            pltpu.SemaphoreType.DMA,
            pltpu.SemaphoreType.DMA,
            pltpu.SemaphoreType.DMA,
        ],
    )
    def gather_kernel(idx_hbm, table_hbm, out_hbm, idx_v, ra, rb,
                      sga, sgb, ssa, ssb):
        wid = lax.axis_index("s") * info.num_cores + lax.axis_index("c")
        base = wid * rows_per_w
        pltpu.sync_copy(idx_hbm.at[:, pl.ds(base, rows_per_w)], idx_v)

        def gath(h, buf, sem):
            return pltpu.async_copy(table_hbm.at[idx_v.at[h]], buf, sem)

        def stor(h, buf, sem):
            return pltpu.async_copy(
                buf, out_hbm.at[h, pl.ds(base, rows_per_w)], sem)

        gath(0, ra, sga)
        gath(1, rb, sgb)

        def body(k, carry):
            ha = 2 * k
            hb = 2 * k + 1
            pltpu.make_async_copy(table_hbm.at[idx_v.at[ha]], ra, sga).wait()
            stor(ha, ra, ssa)
            pltpu.make_async_copy(table_hbm.at[idx_v.at[hb]], rb, sgb).wait()
            stor(hb, rb, ssb)
            pltpu.make_async_copy(
                ra, out_hbm.at[ha, pl.ds(base, rows_per_w)], ssa).wait()
            gath(ha + 2, ra, sga)
            pltpu.make_async_copy(
                rb, out_hbm.at[hb, pl.ds(base, rows_per_w)], ssb).wait()
            gath(hb + 2, rb, sgb)
            return carry

        last = hist // 2 - 1
        lax.fori_loop(0, last, body, 0)
        ha = 2 * last
        hb = 2 * last + 1
        pltpu.make_async_copy(table_hbm.at[idx_v.at[ha]], ra, sga).wait()
        stor(ha, ra, ssa)
        pltpu.make_async_copy(table_hbm.at[idx_v.at[hb]], rb, sgb).wait()
        stor(hb, rb, ssb)
        pltpu.make_async_copy(
            ra, out_hbm.at[ha, pl.ds(base, rows_per_w)], ssa).wait()
        pltpu.make_async_copy(
            rb, out_hbm.at[hb, pl.ds(base, rows_per_w)], ssb).wait()

    return gather_kernel


def kernel(word_ids, sequences, sequences_length, char_embed, W, b):
    batch, hist = word_ids.shape
    vocab = sequences.shape[0]
    ids = jnp.asarray(word_ids, jnp.int32)
    seqs = jnp.asarray(sequences, jnp.int32)
    lens = jnp.asarray(sequences_length, jnp.int32).reshape(
        vocab // V_BLOCK, 1, V_BLOCK)
    table = _make_table_builder(vocab)(
        seqs, lens, char_embed, W, b.reshape(1, EMBED_SIZE))
    out_t = _make_gather(batch, hist, vocab)(ids.T, table)
    return jnp.transpose(out_t, (1, 0, 2))


# speedup vs baseline: 1.0164x; 1.0164x over previous
import functools

import jax
import jax.numpy as jnp
from jax import lax
from jax.experimental import pallas as pl
from jax.experimental.pallas import tpu as pltpu
from jax.experimental.pallas import tpu_sc as plsc

MAX_WORD_LEN = 16
CHAR_VOCAB = 128
CHAR_DIM = 64
EMBED_SIZE = 128

V_BLOCK = 4000


def _table_body(seqs_ref, lens_ref, ce_ref, w_ref, b_ref, out_ref):
    L = MAX_WORD_LEN
    seqs = seqs_ref[...]
    lens = lens_ref[...].reshape(V_BLOCK, 1)
    lens_f = lens.astype(jnp.float32)
    ones_l = jnp.ones((1, L), jnp.float32)
    lens_b = jnp.dot(lens_f, ones_l,
                     preferred_element_type=jnp.float32)
    pos = lax.broadcasted_iota(jnp.int32, (1, L), 1).astype(jnp.float32)
    sm = jnp.where(pos < lens_b, seqs.astype(jnp.float32),
                   float(CHAR_VOCAB)).astype(jnp.bfloat16)
    col = lax.broadcasted_iota(jnp.int32, (L, L * CHAR_VOCAB), 1)
    row = lax.broadcasted_iota(jnp.int32, (L, L * CHAR_VOCAB), 0)
    p = (col // CHAR_VOCAB == row).astype(jnp.bfloat16)
    ch_all = jnp.dot(sm, p,
                     preferred_element_type=jnp.float32)
    cvec = lax.broadcasted_iota(
        jnp.int32, (1, CHAR_VOCAB), 1).astype(jnp.float32)
    counts_bf = jnp.zeros((V_BLOCK, CHAR_VOCAB), jnp.bfloat16)
    for l in range(L):
        sl = ch_all[:, l * CHAR_VOCAB:(l + 1) * CHAR_VOCAB]
        counts_bf = counts_bf + (sl == cvec).astype(jnp.bfloat16)
    counts = counts_bf.astype(jnp.float32)
    inv1 = 1.0 / lens_f
    inv_b = jnp.dot(inv1, jnp.ones((1, CHAR_DIM), jnp.float32),
                    preferred_element_type=jnp.float32)
    pooled = jnp.dot(counts, ce_ref[...],
                     preferred_element_type=jnp.float32) * inv_b
    out_ref[...] = jnp.dot(pooled, w_ref[...],
                           preferred_element_type=jnp.float32) + b_ref[...]


@functools.lru_cache(maxsize=None)
def _make_table_builder(vocab):
    grid = vocab // V_BLOCK
    return pl.pallas_call(
        _table_body,
        grid=(grid,),
        in_specs=[
            pl.BlockSpec((V_BLOCK, MAX_WORD_LEN), lambda i: (i, 0)),
            pl.BlockSpec((1, 1, V_BLOCK), lambda i: (i, 0, 0)),
            pl.BlockSpec((CHAR_VOCAB, CHAR_DIM), lambda i: (0, 0)),
            pl.BlockSpec((CHAR_DIM, EMBED_SIZE), lambda i: (0, 0)),
            pl.BlockSpec((1, EMBED_SIZE), lambda i: (0, 0)),
        ],
        out_specs=pl.BlockSpec((V_BLOCK, EMBED_SIZE), lambda i: (i, 0)),
        out_shape=jax.ShapeDtypeStruct((vocab, EMBED_SIZE), jnp.float32),
    )


@functools.lru_cache(maxsize=None)
def _make_gather(batch, hist, vocab):
    info = plsc.get_sparse_core_info()
    nw = info.num_cores * info.num_subcores
    group = 5 if hist % 5 == 0 else 1
    n_groups = hist // group
    rows_per_w = batch // nw
    assert batch % nw == 0 and rows_per_w % 8 == 0
    mesh = plsc.VectorSubcoreMesh(core_axis_name="c", subcore_axis_name="s")

    @functools.partial(
        pl.kernel,
        mesh=mesh,
        out_type=jax.ShapeDtypeStruct((hist, batch, EMBED_SIZE), jnp.float32),
        scratch_types=[
            pltpu.VMEM((hist, rows_per_w), jnp.int32),
            pltpu.VMEM((group, rows_per_w, EMBED_SIZE), jnp.float32),
            pltpu.SemaphoreType.DMA,
        ],
    )
    def gather_kernel(idx_hbm, table_hbm, out_hbm, idx_v, rows_v, sem):
        wid = lax.axis_index("s") * info.num_cores + lax.axis_index("c")
        base = wid * rows_per_w
        pltpu.sync_copy(idx_hbm.at[:, pl.ds(base, rows_per_w)], idx_v)

        def group_body(g, carry):
            copies = [
                pltpu.async_copy(
                    table_hbm.at[idx_v.at[g * group + j]], rows_v.at[j], sem)
                for j in range(group)
            ]
            for c in copies:
                c.wait()
            pltpu.sync_copy(
                rows_v,
                out_hbm.at[pl.ds(g * group, group), pl.ds(base, rows_per_w)])
            return carry

        lax.fori_loop(0, n_groups, group_body, 0)

    return gather_kernel


def kernel(word_ids, sequences, sequences_length, char_embed, W, b):
    batch, hist = word_ids.shape
    vocab = sequences.shape[0]
    ids = jnp.asarray(word_ids, jnp.int32)
    seqs = jnp.asarray(sequences, jnp.int32)
    lens = jnp.asarray(sequences_length, jnp.int32).reshape(
        vocab // V_BLOCK, 1, V_BLOCK)
    table = _make_table_builder(vocab)(
        seqs, lens, char_embed, W, b.reshape(1, EMBED_SIZE))
    out_t = _make_gather(batch, hist, vocab)(ids.T, table)
    return jnp.transpose(out_t, (1, 0, 2))
